# Initial kernel scaffold; baseline (speedup 1.0000x reference)
import jax
import jax.numpy as jnp
from jax.experimental import pallas as pl


def kernel(x, smi_em, edge_index, batch, Wih_f, Whh_f, bih_f, bhh_f, Wih_r, Whh_r, bih_r, bhh_r, fc1_w, fc1_b, fc2_w, fc2_b, lin1_w, lin1_b, lin2_w, lin2_b, g1_w, g1_b, g2_w, g2_b, f1_w, f1_b, f2_w, f2_b, c1_w, c1_b, c2_w, c2_b):
    raise NotImplementedError("write your pallas kernel here")


# speedup vs baseline: 2.0389x; 2.0389x over previous
import functools

import jax
import jax.numpy as jnp
from jax import lax
from jax.experimental import pallas as pl
from jax.experimental.pallas import tpu as pltpu
from jax.experimental.pallas import tpu_sc as plsc

N_NODES = 10000
N_EDGES = 320000
D_FEAT = 128
N_GRAPHS = 64
AUG_RATIO = 0.4

_NS = 16
_ROWS_PT = 632
_NP = _NS * _ROWS_PT
_CHUNK = 128
_GRP = 32
_NGRP = 5
_CHUNKS_PT = _GRP * _NGRP
_EPT = _CHUNKS_PT * _CHUNK


def _seg_body(xt_hbm, yt_hbm, src_hbm, dst_hbm, aggx_hbm, aggy_hbm,
              src_v, dst_v, rows_v, sem, acc_sh):
    c = lax.axis_index("c")
    s = lax.axis_index("s")
    row0 = s * _ROWS_PT

    @pl.when(c == 0)
    def _():
        pltpu.sync_copy(xt_hbm.at[pl.ds(row0, _ROWS_PT)],
                        acc_sh.at[pl.ds(row0, _ROWS_PT)])

    @pl.when(c == 1)
    def _():
        pltpu.sync_copy(yt_hbm.at[pl.ds(row0, _ROWS_PT)],
                        acc_sh.at[pl.ds(row0, _ROWS_PT)])

    plsc.subcore_barrier()

    def group(g, carry):
        pltpu.sync_copy(src_hbm.at[s, pl.ds(g * _GRP, _GRP)], src_v)
        pltpu.sync_copy(dst_hbm.at[s, pl.ds(g * _GRP, _GRP)], dst_v)

        def body(j, carry2):
            @pl.when(c == 0)
            def _():
                pltpu.async_copy(xt_hbm.at[src_v.at[j]], rows_v, sem).wait()

            @pl.when(c == 1)
            def _():
                pltpu.async_copy(yt_hbm.at[src_v.at[j]], rows_v, sem).wait()

            pltpu.sync_copy(rows_v, acc_sh.at[dst_v.at[j]], add=True)
            return carry2

        return lax.fori_loop(0, _GRP, body, carry)

    lax.fori_loop(0, _NGRP, group, 0)
    plsc.subcore_barrier()

    @pl.when(c == 0)
    def _():
        pltpu.sync_copy(acc_sh.at[pl.ds(row0, _ROWS_PT)],
                        aggx_hbm.at[pl.ds(row0, _ROWS_PT)])

    @pl.when(c == 1)
    def _():
        pltpu.sync_copy(acc_sh.at[pl.ds(row0, _ROWS_PT)],
                        aggy_hbm.at[pl.ds(row0, _ROWS_PT)])


_seg_call = pl.kernel(
    _seg_body,
    out_type=[jax.ShapeDtypeStruct((_NP, D_FEAT), jnp.float32)] * 2,
    mesh=plsc.VectorSubcoreMesh(core_axis_name="c", subcore_axis_name="s"),
    scratch_types=[
        pltpu.VMEM((_GRP, _CHUNK), jnp.int32),
        pltpu.VMEM((_GRP, _CHUNK), jnp.int32),
        pltpu.VMEM((_CHUNK, D_FEAT), jnp.float32),
        pltpu.SemaphoreType.DMA,
        pltpu.VMEM_SHARED((_NP, D_FEAT), jnp.float32),
    ],
)


def _gru_dir(seq, Wih, Whh, bih, bhh):
    H = Whh.shape[1]
    h0 = jnp.zeros((seq.shape[1], H), dtype=seq.dtype)

    def step(h, x_t):
        gi = x_t @ Wih.T + bih
        gh = h @ Whh.T + bhh
        ir, iz, inn = jnp.split(gi, 3, axis=-1)
        hr, hz, hn = jnp.split(gh, 3, axis=-1)
        r = jax.nn.sigmoid(ir + hr)
        z = jax.nn.sigmoid(iz + hz)
        n = jnp.tanh(inn + r * hn)
        h_new = (1.0 - z) * n + z * h
        return h_new, h_new

    _, out = jax.lax.scan(step, h0, seq)
    return out


def kernel(x, smi_em, edge_index, batch, Wih_f, Whh_f, bih_f, bhh_f,
           Wih_r, Whh_r, bih_r, bhh_r, fc1_w, fc1_b, fc2_w, fc2_b,
           lin1_w, lin1_b, lin2_w, lin2_b, g1_w, g1_b, g2_w, g2_b,
           f1_w, f1_b, f2_w, f2_b, c1_w, c1_b, c2_w, c2_b):
    s = smi_em.reshape(-1, 100, 100)
    fwd = _gru_dir(s, Wih_f, Whh_f, bih_f, bhh_f)
    rev = _gru_dir(s[::-1], Wih_r, Whh_r, bih_r, bhh_r)[::-1]
    h = jax.nn.relu(jnp.concatenate([fwd, rev], axis=-1))
    t = jax.nn.relu(h @ fc1_w.T + fc1_b) @ fc2_w.T + fc2_b
    att = jax.nn.softmax(jnp.tanh(t), axis=1)
    smi = jnp.sum(jnp.matmul(att.transpose(0, 2, 1), h), axis=1) / 10.0
    smi = (smi @ lin1_w.T + lin1_b) @ lin2_w.T + lin2_b

    src = edge_index[0].astype(jnp.int32)
    dst = edge_index[1].astype(jnp.int32)
    pad = _NS * _EPT - N_EDGES
    src_p = jnp.concatenate([src, jnp.zeros((pad,), jnp.int32)])
    dst_p = jnp.concatenate([dst, jnp.full((pad,), N_NODES, jnp.int32)])
    src3 = src_p.reshape(_NS, _CHUNKS_PT, _CHUNK)
    dst3 = dst_p.reshape(_NS, _CHUNKS_PT, _CHUNK)

    keep = (jax.random.uniform(jax.random.key(42), (N_NODES,)) < AUG_RATIO
            ).astype(x.dtype)
    y = x * keep[:, None]
    zpad = jnp.zeros((_NP - N_NODES, D_FEAT), jnp.float32)
    x_t = jnp.concatenate([x, zpad])
    y_t = jnp.concatenate([y, zpad])

    sx1, sy1 = _seg_call(x_t, y_t, src3, dst3)
    h1x = jax.nn.relu(sx1 @ c1_w.T + c1_b)
    h1y = jax.nn.relu(sy1 @ c1_w.T + c1_b)

    sx2, sy2 = _seg_call(h1x, h1y, src3, dst3)
    h2x = jax.nn.relu(sx2[:N_NODES] @ c2_w.T + c2_b)
    h2y = jax.nn.relu(sy2[:N_NODES] @ c2_w.T + c2_b)

    def pool_mlp(h2):
        gmax = jax.ops.segment_max(h2, batch, num_segments=N_GRAPHS)
        ssum = jax.ops.segment_sum(h2, batch, num_segments=N_GRAPHS)
        cnt = jax.ops.segment_sum(jnp.ones((N_NODES, 1), h2.dtype), batch,
                                  num_segments=N_GRAPHS)
        gmean = ssum / jnp.maximum(cnt, 1.0)
        g = jnp.concatenate([gmax, gmean], axis=1)
        return jax.nn.relu(g @ g1_w.T + g1_b) @ g2_w.T + g2_b

    x_g = pool_mlp(h2x)
    y_g = pool_mlp(h2y)

    z = jnp.concatenate([x_g, smi], axis=1)
    z = jax.nn.relu(z @ f1_w.T + f1_b) @ f2_w.T + f2_b
    return (z, x_g, y_g)
